# Initial kernel scaffold; baseline (speedup 1.0000x reference)
import functools

import jax
import jax.numpy as jnp
from jax import lax
from jax.experimental import pallas as pl
from jax.experimental.pallas import tpu as pltpu
from jax.experimental.pallas import tpu_sc as plsc

N = 10000
E = 320000
D = 128
DA = 144
NC, NS = 2, 16
NW = NC * NS
CHUNK = 128
CPW = 79
E_PAD = NW * CPW * CHUNK
ROWS_PER_TILE = N // NS
NPAD = N + 16


def _make_sc_agg(width: int):
    mesh = plsc.VectorSubcoreMesh(
        core_axis_name="c", subcore_axis_name="s", num_cores=NC, num_subcores=NS
    )

    @functools.partial(
        pl.kernel,
        out_type=jax.ShapeDtypeStruct((NC, N, width), jnp.float32),
        mesh=mesh,
        scratch_types=[
            pltpu.VMEM((CHUNK,), jnp.int32),
            pltpu.VMEM((CHUNK,), jnp.int32),
            pltpu.VMEM((CHUNK, width), jnp.float32),
            pltpu.VMEM_SHARED((N, width), jnp.float32),
            pltpu.SemaphoreType.DMA,
        ],
    )
    def sc_agg(table_hbm, src_hbm, dst_hbm, zrows_hbm, out_hbm,
               sidx, didx, rows, acc, sem):
        c = lax.axis_index("c")
        s = lax.axis_index("s")
        wid = s * NC + c
        r0 = s * ROWS_PER_TILE
        pltpu.sync_copy(zrows_hbm, acc.at[pl.ds(r0, ROWS_PER_TILE)])
        plsc.subcore_barrier()

        def body(j, carry):
            ch = wid * CPW + j
            pltpu.sync_copy(src_hbm.at[ch], sidx)
            pltpu.sync_copy(dst_hbm.at[ch], didx)
            pltpu.async_copy(table_hbm.at[sidx], rows, sem).wait()
            pltpu.sync_copy(rows, acc.at[didx], add=True)
            return carry

        lax.fori_loop(0, CPW, body, 0)
        plsc.subcore_barrier()
        pltpu.sync_copy(acc.at[pl.ds(r0, ROWS_PER_TILE)],
                        out_hbm.at[c, pl.ds(r0, ROWS_PER_TILE)])

    return sc_agg


_sc_agg1 = _make_sc_agg(DA)
_sc_agg2 = _make_sc_agg(D)


def _tc1_body(acc_ref, x_ref, w1_ref, wr1_ref, b1_ref, h_ref, invdeg_ref):
    s = acc_ref[0, :, :D] + acc_ref[1, :, :D]
    deg = acc_ref[0, :, D:D + 1] + acc_ref[1, :, D:D + 1]
    inv = 1.0 / jnp.maximum(deg, 1.0)
    agg = jnp.dot(s, w1_ref[...], preferred_element_type=jnp.float32)
    root = jnp.dot(x_ref[...], wr1_ref[...], preferred_element_type=jnp.float32)
    h_ref[...] = jnp.maximum(agg * inv + root + b1_ref[...], 0.0)
    invdeg_ref[...] = inv


def _tc2_body(acc_ref, h_ref, invdeg_ref, w2_ref, wr2_ref, b2_ref, out_ref):
    s = acc_ref[0] + acc_ref[1]
    agg = jnp.dot(s, w2_ref[...], preferred_element_type=jnp.float32)
    root = jnp.dot(h_ref[...], wr2_ref[...], preferred_element_type=jnp.float32)
    out_ref[...] = agg * invdeg_ref[...] + root + b2_ref[...]


_R = 1000


def _tc_layer1(acc, x, W1, Wr1, b1):
    grid = (N // _R,)
    return pl.pallas_call(
        _tc1_body,
        grid=grid,
        in_specs=[
            pl.BlockSpec((NC, _R, DA), lambda i: (0, i, 0)),
            pl.BlockSpec((_R, D), lambda i: (i, 0)),
            pl.BlockSpec((D, D), lambda i: (0, 0)),
            pl.BlockSpec((D, D), lambda i: (0, 0)),
            pl.BlockSpec((1, D), lambda i: (0, 0)),
        ],
        out_specs=[
            pl.BlockSpec((_R, D), lambda i: (i, 0)),
            pl.BlockSpec((_R, 1), lambda i: (i, 0)),
        ],
        out_shape=[
            jax.ShapeDtypeStruct((N, D), jnp.float32),
            jax.ShapeDtypeStruct((N, 1), jnp.float32),
        ],
    )(acc, x, W1, Wr1, b1)


def _tc_layer2(acc, h, invdeg, W2, Wr2, b2):
    grid = (N // _R,)
    return pl.pallas_call(
        _tc2_body,
        grid=grid,
        in_specs=[
            pl.BlockSpec((NC, _R, D), lambda i: (0, i, 0)),
            pl.BlockSpec((_R, D), lambda i: (i, 0)),
            pl.BlockSpec((_R, 1), lambda i: (i, 0)),
            pl.BlockSpec((D, D), lambda i: (0, 0)),
            pl.BlockSpec((D, D), lambda i: (0, 0)),
            pl.BlockSpec((1, D), lambda i: (0, 0)),
        ],
        out_specs=pl.BlockSpec((_R, D), lambda i: (i, 0)),
        out_shape=jax.ShapeDtypeStruct((N, D), jnp.float32),
    )(acc, h, invdeg, W2, Wr2, b2)


def kernel(x, edge_index, W1, Wr1, b1, W2, Wr2, b2):
    src = edge_index[0].astype(jnp.int32)
    dst = edge_index[1].astype(jnp.int32)
    npad = E_PAD - E
    src_p = jnp.concatenate([src, jnp.full((npad,), N, jnp.int32)]).reshape(-1, CHUNK)
    dst_p = jnp.concatenate([dst, jnp.zeros((npad,), jnp.int32)]).reshape(-1, CHUNK)

    x_aug = jnp.zeros((NPAD, DA), jnp.float32)
    x_aug = x_aug.at[:N, :D].set(x).at[:N, D].set(1.0)
    z1 = jnp.zeros((ROWS_PER_TILE, DA), jnp.float32)
    z2 = jnp.zeros((ROWS_PER_TILE, D), jnp.float32)

    acc1 = _sc_agg1(x_aug, src_p, dst_p, z1)
    h, invdeg = _tc_layer1(acc1, x, W1, Wr1, b1.reshape(1, D))

    h_pad = jnp.concatenate([h, jnp.zeros((NPAD - N, D), jnp.float32)])
    acc2 = _sc_agg2(h_pad, src_p, dst_p, z2)
    out = _tc_layer2(acc2, h, invdeg, W2, Wr2, b2.reshape(1, D))
    return out.reshape(-1)


# speedup vs baseline: 3.8737x; 3.8737x over previous
import functools

import jax
import jax.numpy as jnp
from jax import lax
from jax.experimental import pallas as pl
from jax.experimental.pallas import tpu as pltpu
from jax.experimental.pallas import tpu_sc as plsc

N = 10000
E = 320000
D = 128
DW = 16
NC, NS = 2, 16
NW = NC * NS
CHUNK = 128
CPW = 80
E_PAD = NW * CPW * CHUNK
ROWS_PER_TILE = 632
NACC = NS * ROWS_PER_TILE


@functools.lru_cache(maxsize=None)
def _make_sc_agg(with_deg: bool):
    mesh = plsc.VectorSubcoreMesh(
        core_axis_name="c", subcore_axis_name="s", num_cores=NC, num_subcores=NS
    )
    out_type = [jax.ShapeDtypeStruct((NC, NACC, D), jnp.float32)]
    scratch = [
        pltpu.VMEM((CHUNK,), jnp.int32),
        pltpu.VMEM((CHUNK,), jnp.int32),
        pltpu.VMEM((CHUNK,), jnp.int32),
        pltpu.VMEM((CHUNK,), jnp.int32),
        pltpu.VMEM((CHUNK, D), jnp.float32),
        pltpu.VMEM((CHUNK, D), jnp.float32),
        pltpu.VMEM_SHARED((NACC, D), jnp.float32),
        pltpu.SemaphoreType.DMA,
        pltpu.SemaphoreType.DMA,
    ]
    if with_deg:
        out_type.append(jax.ShapeDtypeStruct((NC, NACC, DW), jnp.float32))
        scratch += [
            pltpu.VMEM((CHUNK, DW), jnp.float32),
            pltpu.VMEM_SHARED((NACC, DW), jnp.float32),
        ]

    @functools.partial(
        pl.kernel,
        out_type=tuple(out_type),
        mesh=mesh,
        scratch_types=tuple(scratch),
        compiler_params=pltpu.CompilerParams(use_tc_tiling_on_sc=False),
    )
    def sc_agg(table_hbm, src_hbm, dst_hbm, zrows_hbm, *rest):
        if with_deg:
            (ones_hbm, zdeg_hbm, out_hbm, deg_hbm,
             sidx0, sidx1, didx0, didx1, rows0, rows1, acc, sem0, sem1,
             onesbuf, dacc) = rest
        else:
            (out_hbm,
             sidx0, sidx1, didx0, didx1, rows0, rows1, acc, sem0, sem1) = rest
        c = lax.axis_index("c")
        s = lax.axis_index("s")
        wid = s * NC + c
        r0 = s * ROWS_PER_TILE
        base = wid * CPW
        sidx = (sidx0, sidx1)
        didx = (didx0, didx1)
        rows = (rows0, rows1)
        sem = (sem0, sem1)

        pltpu.sync_copy(zrows_hbm, acc.at[pl.ds(r0, ROWS_PER_TILE)])
        if with_deg:
            pltpu.sync_copy(ones_hbm, onesbuf)
            pltpu.sync_copy(zdeg_hbm, dacc.at[pl.ds(r0, ROWS_PER_TILE)])
        plsc.subcore_barrier()

        def start(ch, b):
            pltpu.sync_copy(src_hbm.at[ch], sidx[b])
            pltpu.sync_copy(dst_hbm.at[ch], didx[b])
            pltpu.async_copy(table_hbm.at[sidx[b]], rows[b], sem[b])

        def finish(b):
            pltpu.make_async_copy(table_hbm.at[sidx[b]], rows[b], sem[b]).wait()
            pltpu.sync_copy(rows[b], acc.at[didx[b]], add=True)
            if with_deg:
                pltpu.sync_copy(onesbuf, dacc.at[didx[b]], add=True)

        start(base, 0)

        def body(k, carry):
            start(base + 2 * k + 1, 1)
            finish(0)

            @pl.when(2 * k + 2 < CPW)
            def _():
                start(base + 2 * k + 2, 0)

            finish(1)
            return carry

        lax.fori_loop(0, CPW // 2, body, 0)
        plsc.subcore_barrier()
        pltpu.sync_copy(acc.at[pl.ds(r0, ROWS_PER_TILE)],
                        out_hbm.at[c, pl.ds(r0, ROWS_PER_TILE)])
        if with_deg:
            pltpu.sync_copy(dacc.at[pl.ds(r0, ROWS_PER_TILE)],
                            deg_hbm.at[c, pl.ds(r0, ROWS_PER_TILE)])

    return sc_agg


def _tc1_body(acc_ref, deg_ref, x_ref, w1_ref, wr1_ref, b1_ref,
              h_ref, invdeg_ref):
    s = acc_ref[0] + acc_ref[1]
    deg = deg_ref[0, :, 0:1] + deg_ref[1, :, 0:1]
    inv = 1.0 / jnp.maximum(deg, 1.0)
    agg = jnp.dot(s, w1_ref[...], preferred_element_type=jnp.float32)
    root = jnp.dot(x_ref[...], wr1_ref[...], preferred_element_type=jnp.float32)
    h_ref[...] = jnp.maximum(agg * inv + root + b1_ref[...], 0.0)
    invdeg_ref[...] = inv


def _tc2_body(acc_ref, h_ref, invdeg_ref, w2_ref, wr2_ref, b2_ref, out_ref):
    s = acc_ref[0] + acc_ref[1]
    agg = jnp.dot(s, w2_ref[...], preferred_element_type=jnp.float32)
    root = jnp.dot(h_ref[...], wr2_ref[...], preferred_element_type=jnp.float32)
    out_ref[...] = agg * invdeg_ref[...] + root + b2_ref[...]


_R = 1000


def _tc_layer1(acc, deg, x, W1, Wr1, b1):
    return pl.pallas_call(
        _tc1_body,
        grid=(N // _R,),
        in_specs=[
            pl.BlockSpec((NC, _R, D), lambda i: (0, i, 0)),
            pl.BlockSpec((NC, _R, DW), lambda i: (0, i, 0)),
            pl.BlockSpec((_R, D), lambda i: (i, 0)),
            pl.BlockSpec((D, D), lambda i: (0, 0)),
            pl.BlockSpec((D, D), lambda i: (0, 0)),
            pl.BlockSpec((1, D), lambda i: (0, 0)),
        ],
        out_specs=[
            pl.BlockSpec((_R, D), lambda i: (i, 0)),
            pl.BlockSpec((_R, 1), lambda i: (i, 0)),
        ],
        out_shape=[
            jax.ShapeDtypeStruct((N, D), jnp.float32),
            jax.ShapeDtypeStruct((N, 1), jnp.float32),
        ],
    )(acc, deg, x, W1, Wr1, b1)


def _tc_layer2(acc, h, invdeg, W2, Wr2, b2):
    return pl.pallas_call(
        _tc2_body,
        grid=(N // _R,),
        in_specs=[
            pl.BlockSpec((NC, _R, D), lambda i: (0, i, 0)),
            pl.BlockSpec((_R, D), lambda i: (i, 0)),
            pl.BlockSpec((_R, 1), lambda i: (i, 0)),
            pl.BlockSpec((D, D), lambda i: (0, 0)),
            pl.BlockSpec((D, D), lambda i: (0, 0)),
            pl.BlockSpec((1, D), lambda i: (0, 0)),
        ],
        out_specs=pl.BlockSpec((_R, D), lambda i: (i, 0)),
        out_shape=jax.ShapeDtypeStruct((N, D), jnp.float32),
    )(acc, h, invdeg, W2, Wr2, b2)


def kernel(x, edge_index, W1, Wr1, b1, W2, Wr2, b2):
    src = edge_index[0].astype(jnp.int32)
    dst = edge_index[1].astype(jnp.int32)
    npad = E_PAD - E
    src_p = jnp.concatenate([src, jnp.zeros((npad,), jnp.int32)]).reshape(-1, CHUNK)
    dst_p = jnp.concatenate([dst, jnp.full((npad,), N, jnp.int32)]).reshape(-1, CHUNK)

    z_main = jnp.zeros((ROWS_PER_TILE, D), jnp.float32)
    z_deg = jnp.zeros((ROWS_PER_TILE, DW), jnp.float32)
    ones_c = jnp.ones((CHUNK, DW), jnp.float32)

    acc1, deg = _make_sc_agg(True)(x, src_p, dst_p, z_main, ones_c, z_deg)
    h, invdeg = _tc_layer1(acc1, deg, x, W1, Wr1, b1.reshape(1, D))

    acc2, = _make_sc_agg(False)(h, src_p, dst_p, z_main)
    out = _tc_layer2(acc2, h, invdeg, W2, Wr2, b2.reshape(1, D))
    return out.reshape(-1)


# speedup vs baseline: 3.9110x; 1.0096x over previous
import functools

import jax
import jax.numpy as jnp
from jax import lax
from jax.experimental import pallas as pl
from jax.experimental.pallas import tpu as pltpu
from jax.experimental.pallas import tpu_sc as plsc

N = 10000
E = 320000
D = 128
DW = 16
NC, NS = 2, 16
NW = NC * NS
CHUNK = 128
CPW = 80
NBUF = 2
ISLOT = 4
E_PAD = NW * CPW * CHUNK
ROWS_PER_TILE = 632
NACC = NS * ROWS_PER_TILE


@functools.lru_cache(maxsize=None)
def _make_sc_agg(with_deg: bool):
    mesh = plsc.VectorSubcoreMesh(
        core_axis_name="c", subcore_axis_name="s", num_cores=NC, num_subcores=NS
    )
    out_type = [jax.ShapeDtypeStruct((NC, NACC, D), jnp.float32)]
    scratch = []
    scratch += [pltpu.VMEM((CHUNK,), jnp.int32) for _ in range(ISLOT)]
    scratch += [pltpu.VMEM((CHUNK,), jnp.int32) for _ in range(ISLOT)]
    scratch += [pltpu.VMEM((CHUNK, D), jnp.float32) for _ in range(NBUF)]
    scratch += [
        pltpu.VMEM_SHARED((NACC, D), jnp.float32),
    ]
    scratch += [pltpu.SemaphoreType.DMA for _ in range(ISLOT)]
    scratch += [pltpu.SemaphoreType.DMA for _ in range(NBUF)]
    scratch += [pltpu.SemaphoreType.DMA for _ in range(NBUF)]
    if with_deg:
        out_type.append(jax.ShapeDtypeStruct((NC, NACC, DW), jnp.float32))
        scratch += [
            pltpu.VMEM((CHUNK, DW), jnp.float32),
            pltpu.VMEM_SHARED((NACC, DW), jnp.float32),
        ]

    @functools.partial(
        pl.kernel,
        out_type=tuple(out_type),
        mesh=mesh,
        scratch_types=tuple(scratch),
        compiler_params=pltpu.CompilerParams(use_tc_tiling_on_sc=False),
    )
    def sc_agg(table_hbm, src_hbm, dst_hbm, zrows_hbm, *rest):
        if with_deg:
            (ones_hbm, zdeg_hbm, out_hbm, deg_hbm, *rest2) = rest
        else:
            (out_hbm, *rest2) = rest
        sidx = rest2[:ISLOT]
        didx = rest2[ISLOT:2 * ISLOT]
        rows = rest2[2 * ISLOT:2 * ISLOT + NBUF]
        p = 2 * ISLOT + NBUF
        acc = rest2[p]
        isem = rest2[p + 1:p + 1 + ISLOT]
        gsem = rest2[p + 1 + ISLOT:p + 1 + ISLOT + NBUF]
        ssem = rest2[p + 1 + ISLOT + NBUF:p + 1 + ISLOT + 2 * NBUF]
        if with_deg:
            onesbuf, dacc = rest2[p + 1 + ISLOT + 2 * NBUF:]
        c = lax.axis_index("c")
        s = lax.axis_index("s")
        wid = s * NC + c
        r0 = s * ROWS_PER_TILE
        base = wid * CPW

        pltpu.sync_copy(zrows_hbm, acc.at[pl.ds(r0, ROWS_PER_TILE)])
        if with_deg:
            pltpu.sync_copy(ones_hbm, onesbuf)
            pltpu.sync_copy(zdeg_hbm, dacc.at[pl.ds(r0, ROWS_PER_TILE)])
        plsc.subcore_barrier()

        def idx_start(ch, i):
            pltpu.async_copy(src_hbm.at[base + ch], sidx[i], isem[i])
            pltpu.async_copy(dst_hbm.at[base + ch], didx[i], isem[i])

        def idx_wait(ch, i):
            pltpu.make_async_copy(src_hbm.at[base + ch], sidx[i], isem[i]).wait()
            pltpu.make_async_copy(dst_hbm.at[base + ch], didx[i], isem[i]).wait()

        def scat_wait(ch, b, i):
            pltpu.make_async_copy(rows[b], acc.at[didx[i]], ssem[b]).wait()
            if with_deg:
                pltpu.make_async_copy(onesbuf, dacc.at[didx[i]], ssem[b]).wait()

        for ch in range(3):
            idx_start(ch, ch)
        idx_wait(0, 0)
        pltpu.async_copy(table_hbm.at[sidx[0]], rows[0], gsem[0])

        def body(k, carry):
            for o in range(ISLOT):
                ch = ISLOT * k + o
                b = o % NBUF
                bn = (o + 1) % NBUF
                i = o
                inx = (o + 1) % ISLOT

                @pl.when(ch - 1 >= 0)
                def _():
                    scat_wait(ch - 1, bn, (o - 1) % ISLOT)

                @pl.when(ch + 3 < CPW)
                def _():
                    idx_start(ch + 3, (o + 3) % ISLOT)

                @pl.when(ch + 1 < CPW)
                def _():
                    idx_wait(ch + 1, inx)
                    pltpu.async_copy(table_hbm.at[sidx[inx]], rows[bn],
                                     gsem[bn])

                pltpu.make_async_copy(table_hbm.at[sidx[i]], rows[b],
                                      gsem[b]).wait()
                pltpu.async_copy(rows[b], acc.at[didx[i]], ssem[b], add=True)
                if with_deg:
                    pltpu.async_copy(onesbuf, dacc.at[didx[i]], ssem[b],
                                     add=True)
            return carry

        lax.fori_loop(0, CPW // ISLOT, body, 0)
        scat_wait(CPW - 1, (CPW - 1) % NBUF, (CPW - 1) % ISLOT)
        plsc.subcore_barrier()
        pltpu.sync_copy(acc.at[pl.ds(r0, ROWS_PER_TILE)],
                        out_hbm.at[c, pl.ds(r0, ROWS_PER_TILE)])
        if with_deg:
            pltpu.sync_copy(dacc.at[pl.ds(r0, ROWS_PER_TILE)],
                            deg_hbm.at[c, pl.ds(r0, ROWS_PER_TILE)])

    return sc_agg


def _tc1_body(acc_ref, deg_ref, x_ref, w1_ref, wr1_ref, b1_ref,
              h_ref, invdeg_ref):
    s = acc_ref[0] + acc_ref[1]
    deg = deg_ref[0, :, 0:1] + deg_ref[1, :, 0:1]
    inv = 1.0 / jnp.maximum(deg, 1.0)
    agg = jnp.dot(s, w1_ref[...], preferred_element_type=jnp.float32)
    root = jnp.dot(x_ref[...], wr1_ref[...], preferred_element_type=jnp.float32)
    h_ref[...] = jnp.maximum(agg * inv + root + b1_ref[...], 0.0)
    invdeg_ref[...] = inv


def _tc2_body(acc_ref, h_ref, invdeg_ref, w2_ref, wr2_ref, b2_ref, out_ref):
    s = acc_ref[0] + acc_ref[1]
    agg = jnp.dot(s, w2_ref[...], preferred_element_type=jnp.float32)
    root = jnp.dot(h_ref[...], wr2_ref[...], preferred_element_type=jnp.float32)
    out_ref[...] = agg * invdeg_ref[...] + root + b2_ref[...]


_R = 1000


def _tc_layer1(acc, deg, x, W1, Wr1, b1):
    return pl.pallas_call(
        _tc1_body,
        grid=(N // _R,),
        in_specs=[
            pl.BlockSpec((NC, _R, D), lambda i: (0, i, 0)),
            pl.BlockSpec((NC, _R, DW), lambda i: (0, i, 0)),
            pl.BlockSpec((_R, D), lambda i: (i, 0)),
            pl.BlockSpec((D, D), lambda i: (0, 0)),
            pl.BlockSpec((D, D), lambda i: (0, 0)),
            pl.BlockSpec((1, D), lambda i: (0, 0)),
        ],
        out_specs=[
            pl.BlockSpec((_R, D), lambda i: (i, 0)),
            pl.BlockSpec((_R, 1), lambda i: (i, 0)),
        ],
        out_shape=[
            jax.ShapeDtypeStruct((N, D), jnp.float32),
            jax.ShapeDtypeStruct((N, 1), jnp.float32),
        ],
    )(acc, deg, x, W1, Wr1, b1)


def _tc_layer2(acc, h, invdeg, W2, Wr2, b2):
    return pl.pallas_call(
        _tc2_body,
        grid=(N // _R,),
        in_specs=[
            pl.BlockSpec((NC, _R, D), lambda i: (0, i, 0)),
            pl.BlockSpec((_R, D), lambda i: (i, 0)),
            pl.BlockSpec((_R, 1), lambda i: (i, 0)),
            pl.BlockSpec((D, D), lambda i: (0, 0)),
            pl.BlockSpec((D, D), lambda i: (0, 0)),
            pl.BlockSpec((1, D), lambda i: (0, 0)),
        ],
        out_specs=pl.BlockSpec((_R, D), lambda i: (i, 0)),
        out_shape=jax.ShapeDtypeStruct((N, D), jnp.float32),
    )(acc, h, invdeg, W2, Wr2, b2)


def kernel(x, edge_index, W1, Wr1, b1, W2, Wr2, b2):
    src = edge_index[0].astype(jnp.int32)
    dst = edge_index[1].astype(jnp.int32)
    npad = E_PAD - E
    src_p = jnp.concatenate([src, jnp.zeros((npad,), jnp.int32)]).reshape(-1, CHUNK)
    dst_p = jnp.concatenate([dst, jnp.full((npad,), N, jnp.int32)]).reshape(-1, CHUNK)

    z_main = jnp.zeros((ROWS_PER_TILE, D), jnp.float32)
    z_deg = jnp.zeros((ROWS_PER_TILE, DW), jnp.float32)
    ones_c = jnp.ones((CHUNK, DW), jnp.float32)

    acc1, deg = _make_sc_agg(True)(x, src_p, dst_p, z_main, ones_c, z_deg)
    h, invdeg = _tc_layer1(acc1, deg, x, W1, Wr1, b1.reshape(1, D))

    acc2, = _make_sc_agg(False)(h, src_p, dst_p, z_main)
    out = _tc_layer2(acc2, h, invdeg, W2, Wr2, b2.reshape(1, D))
    return out.reshape(-1)


# speedup vs baseline: 4.2995x; 1.0993x over previous
import functools

import jax
import jax.numpy as jnp
from jax import lax
from jax.experimental import pallas as pl
from jax.experimental.pallas import tpu as pltpu
from jax.experimental.pallas import tpu_sc as plsc

N = 10000
E = 320000
D = 128
DW = 16
NC, NS = 2, 16
NW = NC * NS
CHUNK = 128
CPW = 80
NBUF = 2
ISLOT = 4
E_PAD = NW * CPW * CHUNK
ROWS_PER_TILE = 632
NACC = NS * ROWS_PER_TILE


@functools.lru_cache(maxsize=None)
def _make_sc_agg(with_deg: bool):
    mesh = plsc.VectorSubcoreMesh(
        core_axis_name="c", subcore_axis_name="s", num_cores=NC, num_subcores=NS
    )
    out_type = [jax.ShapeDtypeStruct((NC, NACC, D), jnp.float32)]
    scratch = []
    scratch += [pltpu.VMEM((2, CHUNK), jnp.int32) for _ in range(ISLOT)]
    scratch += [pltpu.VMEM((CHUNK, D), jnp.float32) for _ in range(NBUF)]
    scratch += [
        pltpu.VMEM_SHARED((NACC, D), jnp.float32),
    ]
    scratch += [pltpu.SemaphoreType.DMA for _ in range(ISLOT)]
    scratch += [pltpu.SemaphoreType.DMA for _ in range(NBUF)]
    scratch += [pltpu.SemaphoreType.DMA for _ in range(NBUF)]
    if with_deg:
        out_type.append(jax.ShapeDtypeStruct((NC, NACC, DW), jnp.float32))
        scratch += [
            pltpu.VMEM((CHUNK, DW), jnp.float32),
            pltpu.VMEM_SHARED((NACC, DW), jnp.float32),
        ]

    @functools.partial(
        pl.kernel,
        out_type=tuple(out_type),
        mesh=mesh,
        scratch_types=tuple(scratch),
        compiler_params=pltpu.CompilerParams(use_tc_tiling_on_sc=False),
    )
    def sc_agg(table_hbm, ei_hbm, zrows_hbm, *rest):
        if with_deg:
            (ones_hbm, zdeg_hbm, out_hbm, deg_hbm, *rest2) = rest
        else:
            (out_hbm, *rest2) = rest
        ibuf = rest2[:ISLOT]
        rows = rest2[ISLOT:ISLOT + NBUF]
        p = ISLOT + NBUF
        acc = rest2[p]
        isem = rest2[p + 1:p + 1 + ISLOT]
        gsem = rest2[p + 1 + ISLOT:p + 1 + ISLOT + NBUF]
        ssem = rest2[p + 1 + ISLOT + NBUF:p + 1 + ISLOT + 2 * NBUF]
        if with_deg:
            onesbuf, dacc = rest2[p + 1 + ISLOT + 2 * NBUF:]
        c = lax.axis_index("c")
        s = lax.axis_index("s")
        wid = s * NC + c
        r0 = s * ROWS_PER_TILE
        base = wid * CPW

        pltpu.sync_copy(zrows_hbm, acc.at[pl.ds(r0, ROWS_PER_TILE)])
        if with_deg:
            pltpu.sync_copy(ones_hbm, onesbuf)
            pltpu.sync_copy(zdeg_hbm, dacc.at[pl.ds(r0, ROWS_PER_TILE)])
        plsc.subcore_barrier()

        def idx_start(ch, i):
            pltpu.async_copy(ei_hbm.at[base + ch], ibuf[i], isem[i])

        def idx_wait(ch, i):
            pltpu.make_async_copy(ei_hbm.at[base + ch], ibuf[i], isem[i]).wait()

        def scat_wait(ch, b, i):
            pltpu.make_async_copy(rows[b], acc.at[ibuf[i].at[1]], ssem[b]).wait()
            if with_deg:
                pltpu.make_async_copy(onesbuf, dacc.at[ibuf[i].at[1]],
                                      ssem[b]).wait()

        for ch in range(3):
            idx_start(ch, ch)
        idx_wait(0, 0)
        pltpu.async_copy(table_hbm.at[ibuf[0].at[0]], rows[0], gsem[0])

        def body(k, carry):
            for o in range(ISLOT):
                ch = ISLOT * k + o
                b = o % NBUF
                bn = (o + 1) % NBUF
                i = o
                inx = (o + 1) % ISLOT

                @pl.when(ch - 1 >= 0)
                def _():
                    scat_wait(ch - 1, bn, (o - 1) % ISLOT)

                @pl.when(ch + 3 < CPW)
                def _():
                    idx_start(ch + 3, (o + 3) % ISLOT)

                @pl.when(ch + 1 < CPW)
                def _():
                    idx_wait(ch + 1, inx)
                    pltpu.async_copy(table_hbm.at[ibuf[inx].at[0]], rows[bn],
                                     gsem[bn])

                pltpu.make_async_copy(table_hbm.at[ibuf[i].at[0]], rows[b],
                                      gsem[b]).wait()
                pltpu.async_copy(rows[b], acc.at[ibuf[i].at[1]], ssem[b],
                                 add=True)
                if with_deg:
                    pltpu.async_copy(onesbuf, dacc.at[ibuf[i].at[1]], ssem[b],
                                     add=True)
            return carry

        lax.fori_loop(0, CPW // ISLOT, body, 0)
        scat_wait(CPW - 1, (CPW - 1) % NBUF, (CPW - 1) % ISLOT)
        plsc.subcore_barrier()
        pltpu.sync_copy(acc.at[pl.ds(r0, ROWS_PER_TILE)],
                        out_hbm.at[c, pl.ds(r0, ROWS_PER_TILE)])
        if with_deg:
            pltpu.sync_copy(dacc.at[pl.ds(r0, ROWS_PER_TILE)],
                            deg_hbm.at[c, pl.ds(r0, ROWS_PER_TILE)])

    return sc_agg


def _tc1_body(acc_ref, deg_ref, x_ref, w1_ref, wr1_ref, b1_ref,
              h_ref, invdeg_ref):
    s = acc_ref[0] + acc_ref[1]
    deg = deg_ref[0, :, 0:1] + deg_ref[1, :, 0:1]
    inv = 1.0 / jnp.maximum(deg, 1.0)
    agg = jnp.dot(s, w1_ref[...], preferred_element_type=jnp.float32)
    root = jnp.dot(x_ref[...], wr1_ref[...], preferred_element_type=jnp.float32)
    h_ref[...] = jnp.maximum(agg * inv + root + b1_ref[...], 0.0)
    invdeg_ref[...] = inv


def _tc2_body(acc_ref, h_ref, invdeg_ref, w2_ref, wr2_ref, b2_ref, out_ref):
    s = acc_ref[0] + acc_ref[1]
    agg = jnp.dot(s, w2_ref[...], preferred_element_type=jnp.float32)
    root = jnp.dot(h_ref[...], wr2_ref[...], preferred_element_type=jnp.float32)
    out_ref[...] = agg * invdeg_ref[...] + root + b2_ref[...]


_R = 1000


def _tc_layer1(acc, deg, x, W1, Wr1, b1):
    return pl.pallas_call(
        _tc1_body,
        grid=(N // _R,),
        in_specs=[
            pl.BlockSpec((NC, _R, D), lambda i: (0, i, 0)),
            pl.BlockSpec((NC, _R, DW), lambda i: (0, i, 0)),
            pl.BlockSpec((_R, D), lambda i: (i, 0)),
            pl.BlockSpec((D, D), lambda i: (0, 0)),
            pl.BlockSpec((D, D), lambda i: (0, 0)),
            pl.BlockSpec((1, D), lambda i: (0, 0)),
        ],
        out_specs=[
            pl.BlockSpec((_R, D), lambda i: (i, 0)),
            pl.BlockSpec((_R, 1), lambda i: (i, 0)),
        ],
        out_shape=[
            jax.ShapeDtypeStruct((N, D), jnp.float32),
            jax.ShapeDtypeStruct((N, 1), jnp.float32),
        ],
    )(acc, deg, x, W1, Wr1, b1)


def _tc_layer2(acc, h, invdeg, W2, Wr2, b2):
    return pl.pallas_call(
        _tc2_body,
        grid=(N // _R,),
        in_specs=[
            pl.BlockSpec((NC, _R, D), lambda i: (0, i, 0)),
            pl.BlockSpec((_R, D), lambda i: (i, 0)),
            pl.BlockSpec((_R, 1), lambda i: (i, 0)),
            pl.BlockSpec((D, D), lambda i: (0, 0)),
            pl.BlockSpec((D, D), lambda i: (0, 0)),
            pl.BlockSpec((1, D), lambda i: (0, 0)),
        ],
        out_specs=pl.BlockSpec((_R, D), lambda i: (i, 0)),
        out_shape=jax.ShapeDtypeStruct((N, D), jnp.float32),
    )(acc, h, invdeg, W2, Wr2, b2)


def kernel(x, edge_index, W1, Wr1, b1, W2, Wr2, b2):
    src = edge_index[0].astype(jnp.int32)
    dst = edge_index[1].astype(jnp.int32)
    npad = E_PAD - E
    src_p = jnp.concatenate([src, jnp.zeros((npad,), jnp.int32)]).reshape(-1, CHUNK)
    dst_p = jnp.concatenate([dst, jnp.full((npad,), N, jnp.int32)]).reshape(-1, CHUNK)
    ei_p = jnp.stack([src_p, dst_p], axis=1)

    z_main = jnp.zeros((ROWS_PER_TILE, D), jnp.float32)
    z_deg = jnp.zeros((ROWS_PER_TILE, DW), jnp.float32)
    ones_c = jnp.ones((CHUNK, DW), jnp.float32)

    acc1, deg = _make_sc_agg(True)(x, ei_p, z_main, ones_c, z_deg)
    h, invdeg = _tc_layer1(acc1, deg, x, W1, Wr1, b1.reshape(1, D))

    acc2, = _make_sc_agg(False)(h, ei_p, z_main)
    out = _tc_layer2(acc2, h, invdeg, W2, Wr2, b2.reshape(1, D))
    return out.reshape(-1)
